# baseline (speedup 1.0000x reference)
import functools

import jax
import jax.numpy as jnp
from jax import lax
from jax.experimental import pallas as pl
from jax.experimental.pallas import tpu as pltpu
from jax.experimental.pallas import tpu_sc as plsc

_ROWS = 2 * 8192
_COLS = 2048

_SC_ROWS = 4096
_TC_ROWS = _ROWS - _SC_ROWS
_TC_BLOCK_ROWS = 1024

_SC_NC = 2
_SC_NS = 16
_SC_WORKERS = _SC_NC * _SC_NS
_SC_START = _TC_ROWS * _COLS
_N_W = _SC_ROWS * _COLS // _SC_WORKERS
_CHUNK = 16384
_NCHUNK = _N_W // _CHUNK
_UNROLL = 8
_LANES = 16

assert _N_W % _CHUNK == 0 and _NCHUNK % 2 == 0
assert _CHUNK % (_LANES * _UNROLL) == 0


def _tc_kernel(p_ref, t_ref, out_ref, acc_ref):
    i = pl.program_id(0)
    n = pl.num_programs(0)
    p = p_ref[...]
    t = t_ref[...]
    d = p - t
    mask = t != 0.0
    s = jnp.sum(jnp.where(mask, d * d, 0.0))
    c = jnp.sum(jnp.where(mask, 1.0, 0.0))

    @pl.when(i == 0)
    def _init():
        acc_ref[0] = 0.0
        acc_ref[1] = 0.0

    acc_ref[0] += s
    acc_ref[1] += c

    @pl.when(i == n - 1)
    def _fini():
        out_ref[0] = acc_ref[0]
        out_ref[1] = acc_ref[1]


def _sc_body(p_hbm, t_hbm, out_hbm, pbuf, tbuf, obuf, sp0, sp1, st0, st1):
    cc = lax.axis_index("c")
    ss = lax.axis_index("s")
    wid = ss * _SC_NC + cc
    base = _SC_START + wid * _N_W
    psems = (sp0, sp1)
    tsems = (st0, st1)

    def copies(slot, ci):
        off = base + ci * _CHUNK
        return (
            pltpu.make_async_copy(p_hbm.at[pl.ds(off, _CHUNK)], pbuf.at[slot],
                                  psems[slot]),
            pltpu.make_async_copy(t_hbm.at[pl.ds(off, _CHUNK)], tbuf.at[slot],
                                  tsems[slot]),
        )

    def start(slot, ci):
        cp, ct = copies(slot, ci)
        cp.start()
        ct.start()

    def wait(slot, ci):
        cp, ct = copies(slot, ci)
        cp.wait()
        ct.wait()

    zf = jnp.zeros((_LANES,), jnp.float32)

    def chunk_sums(slot, s_accs, c_accs):
        def body(i, carry):
            s_a, c_a = carry
            base_e = i * (_LANES * _UNROLL)
            s_l = list(s_a)
            c_l = list(c_a)
            for u in range(_UNROLL):
                off = base_e + u * _LANES
                pv = pbuf[slot, pl.ds(off, _LANES)]
                tv = tbuf[slot, pl.ds(off, _LANES)]
                m = tv != 0.0
                dm = jnp.where(m, pv - tv, 0.0)
                s_l[u] = s_l[u] + dm * dm
                c_l[u] = c_l[u] + jnp.where(m, 1.0, 0.0)
            return tuple(s_l), tuple(c_l)

        return lax.fori_loop(0, _CHUNK // (_LANES * _UNROLL), body,
                             (s_accs, c_accs))

    s_accs = (zf,) * _UNROLL
    c_accs = (zf,) * _UNROLL

    start(0, 0)

    def outer(g, carry):
        s_a, c_a = carry
        for b in range(2):
            ci = g + b
            nxt = ci + 1

            @pl.when(nxt < _NCHUNK)
            def _():
                start((b + 1) % 2, nxt)

            wait(b, ci)
            s_a, c_a = chunk_sums(b, s_a, c_a)
        return s_a, c_a

    s_accs, c_accs = lax.fori_loop(0, _NCHUNK // 2,
                                   lambda g, cr: outer(g * 2, cr),
                                   (s_accs, c_accs))

    s_tot = s_accs[0]
    for u in range(1, _UNROLL):
        s_tot = s_tot + s_accs[u]
    c_tot = c_accs[0]
    for u in range(1, _UNROLL):
        c_tot = c_tot + c_accs[u]
    obuf[0, :] = s_tot
    obuf[1, :] = c_tot
    pltpu.sync_copy(obuf, out_hbm.at[wid])


_sc_call = functools.partial(
    pl.kernel,
    out_type=jax.ShapeDtypeStruct((_SC_WORKERS, 2, _LANES), jnp.float32),
    mesh=plsc.VectorSubcoreMesh(core_axis_name="c", subcore_axis_name="s"),
    scratch_types=[
        pltpu.VMEM((2, _CHUNK), jnp.float32),
        pltpu.VMEM((2, _CHUNK), jnp.float32),
        pltpu.VMEM((2, _LANES), jnp.float32),
        pltpu.SemaphoreType.DMA,
        pltpu.SemaphoreType.DMA,
        pltpu.SemaphoreType.DMA,
        pltpu.SemaphoreType.DMA,
    ],
)(_sc_body)


def kernel(prediction, target):
    p2 = prediction.reshape(_ROWS, _COLS)
    t2 = target.reshape(_ROWS, _COLS)
    p1 = prediction.reshape(-1)
    t1 = target.reshape(-1)

    tc_out = pl.pallas_call(
        _tc_kernel,
        grid=(_TC_ROWS // _TC_BLOCK_ROWS,),
        in_specs=[
            pl.BlockSpec((_TC_BLOCK_ROWS, _COLS), lambda i: (i, 0)),
            pl.BlockSpec((_TC_BLOCK_ROWS, _COLS), lambda i: (i, 0)),
        ],
        out_specs=pl.BlockSpec(memory_space=pltpu.SMEM),
        out_shape=jax.ShapeDtypeStruct((2,), jnp.float32),
        scratch_shapes=[pltpu.SMEM((2,), jnp.float32)],
    )(p2, t2)

    sc_out = _sc_call(p1, t1)

    s = tc_out[0] + jnp.sum(sc_out[:, 0, :])
    c = tc_out[1] + jnp.sum(sc_out[:, 1, :])
    return s / c


# speedup vs baseline: 2.8272x; 2.8272x over previous
import functools

import jax
import jax.numpy as jnp
from jax import lax
from jax.experimental import pallas as pl
from jax.experimental.pallas import tpu as pltpu
from jax.experimental.pallas import tpu_sc as plsc

_ROWS = 2 * 8192
_COLS = 2048

_SC_ROWS = 4096
_TC_ROWS = _ROWS - _SC_ROWS
_TC_BLOCK_ROWS = 1024

_SC_NC = 2
_SC_NS = 16
_SC_WORKERS = _SC_NC * _SC_NS
_ROWS_W = _SC_ROWS // _SC_WORKERS
_STRIP = 8
_NCHUNK = _ROWS_W // _STRIP
_LANES = 16

assert _ROWS_W % _STRIP == 0 and _NCHUNK % 2 == 0


def _tc_kernel(p_ref, t_ref, out_ref, acc_ref):
    i = pl.program_id(0)
    n = pl.num_programs(0)
    p = p_ref[...]
    t = t_ref[...]
    d = p - t
    mask = t != 0.0
    s = jnp.sum(jnp.where(mask, d * d, 0.0))
    c = jnp.sum(jnp.where(mask, 1.0, 0.0))

    @pl.when(i == 0)
    def _init():
        acc_ref[0] = 0.0
        acc_ref[1] = 0.0

    acc_ref[0] += s
    acc_ref[1] += c

    @pl.when(i == n - 1)
    def _fini():
        out_ref[0] = acc_ref[0]
        out_ref[1] = acc_ref[1]


def _sc_body(p_hbm, t_hbm, out_hbm,
             pbuf0, pbuf1, tbuf0, tbuf1, obuf, sp0, sp1, st0, st1):
    cc = lax.axis_index("c")
    ss = lax.axis_index("s")
    wid = ss * _SC_NC + cc
    row0 = _TC_ROWS + wid * _ROWS_W
    pbufs = (pbuf0, pbuf1)
    tbufs = (tbuf0, tbuf1)
    psems = (sp0, sp1)
    tsems = (st0, st1)

    def copies(slot, ci):
        r = row0 + ci * _STRIP
        return (
            pltpu.make_async_copy(p_hbm.at[pl.ds(r, _STRIP)], pbufs[slot],
                                  psems[slot]),
            pltpu.make_async_copy(t_hbm.at[pl.ds(r, _STRIP)], tbufs[slot],
                                  tsems[slot]),
        )

    def start(slot, ci):
        cp, ct = copies(slot, ci)
        cp.start()
        ct.start()

    def wait(slot, ci):
        cp, ct = copies(slot, ci)
        cp.wait()
        ct.wait()

    zf = jnp.zeros((_LANES,), jnp.float32)

    def chunk_sums(slot, s_accs, c_accs):
        pb = pbufs[slot]
        tb = tbufs[slot]

        def body(i, carry):
            s_a, c_a = carry
            col = i * _LANES
            s_l = list(s_a)
            c_l = list(c_a)
            for r in range(_STRIP):
                pv = pb[r, pl.ds(col, _LANES)]
                tv = tb[r, pl.ds(col, _LANES)]
                m = tv != 0.0
                dm = jnp.where(m, pv - tv, 0.0)
                s_l[r] = s_l[r] + dm * dm
                c_l[r] = c_l[r] + jnp.where(m, 1.0, 0.0)
            return tuple(s_l), tuple(c_l)

        return lax.fori_loop(0, _COLS // _LANES, body, (s_accs, c_accs))

    s_accs = (zf,) * _STRIP
    c_accs = (zf,) * _STRIP

    start(0, 0)

    def outer(g, carry):
        s_a, c_a = carry
        for b in range(2):
            ci = g + b
            nxt = ci + 1

            @pl.when(nxt < _NCHUNK)
            def _():
                start((b + 1) % 2, nxt)

            wait(b, ci)
            s_a, c_a = chunk_sums(b, s_a, c_a)
        return s_a, c_a

    s_accs, c_accs = lax.fori_loop(0, _NCHUNK // 2,
                                   lambda g, cr: outer(g * 2, cr),
                                   (s_accs, c_accs))

    s_tot = s_accs[0]
    c_tot = c_accs[0]
    for r in range(1, _STRIP):
        s_tot = s_tot + s_accs[r]
        c_tot = c_tot + c_accs[r]
    obuf[0, pl.ds(0, _LANES)] = s_tot
    obuf[1, pl.ds(0, _LANES)] = c_tot
    pltpu.sync_copy(obuf, out_hbm.at[wid])


_sc_call = functools.partial(
    pl.kernel,
    out_type=jax.ShapeDtypeStruct((_SC_WORKERS, _STRIP, 128), jnp.float32),
    mesh=plsc.VectorSubcoreMesh(core_axis_name="c", subcore_axis_name="s"),
    scratch_types=[
        pltpu.VMEM((_STRIP, _COLS), jnp.float32),
        pltpu.VMEM((_STRIP, _COLS), jnp.float32),
        pltpu.VMEM((_STRIP, _COLS), jnp.float32),
        pltpu.VMEM((_STRIP, _COLS), jnp.float32),
        pltpu.VMEM((_STRIP, 128), jnp.float32),
        pltpu.SemaphoreType.DMA,
        pltpu.SemaphoreType.DMA,
        pltpu.SemaphoreType.DMA,
        pltpu.SemaphoreType.DMA,
    ],
    compiler_params=pltpu.CompilerParams(use_tc_tiling_on_sc=True),
)(_sc_body)


def kernel(prediction, target):
    p2 = prediction.reshape(_ROWS, _COLS)
    t2 = target.reshape(_ROWS, _COLS)

    tc_out = pl.pallas_call(
        _tc_kernel,
        grid=(_TC_ROWS // _TC_BLOCK_ROWS,),
        in_specs=[
            pl.BlockSpec((_TC_BLOCK_ROWS, _COLS), lambda i: (i, 0)),
            pl.BlockSpec((_TC_BLOCK_ROWS, _COLS), lambda i: (i, 0)),
        ],
        out_specs=pl.BlockSpec(memory_space=pltpu.SMEM),
        out_shape=jax.ShapeDtypeStruct((2,), jnp.float32),
        scratch_shapes=[pltpu.SMEM((2,), jnp.float32)],
    )(p2, t2)

    sc_out = _sc_call(p2, t2)

    s = tc_out[0] + jnp.sum(sc_out[:, 0, :_LANES])
    c = tc_out[1] + jnp.sum(sc_out[:, 1, :_LANES])
    return s / c


# speedup vs baseline: 2.8322x; 1.0018x over previous
import functools

import jax
import jax.numpy as jnp
from jax import lax
from jax.experimental import pallas as pl
from jax.experimental.pallas import tpu as pltpu
from jax.experimental.pallas import tpu_sc as plsc

_ROWS = 2 * 8192
_COLS = 2048

_SC_ROWS = 4096
_TC_ROWS = _ROWS - _SC_ROWS
_TC_BLOCK_ROWS = 1024

_SC_NC = 2
_SC_NS = 16
_SC_WORKERS = _SC_NC * _SC_NS
_ROWS_W = _SC_ROWS // _SC_WORKERS
_STRIP = 8
_NCHUNK = _ROWS_W // _STRIP
_LANES = 16

assert _ROWS_W % _STRIP == 0 and _NCHUNK % 2 == 0


def _tc_kernel(p_ref, t_ref, out_ref, acc_ref):
    i = pl.program_id(0)
    n = pl.num_programs(0)
    p = p_ref[...]
    t = t_ref[...]
    d = p - t
    mask = t != 0.0
    s = jnp.sum(jnp.where(mask, d * d, 0.0))
    c = jnp.sum(jnp.where(mask, 1.0, 0.0))

    @pl.when(i == 0)
    def _init():
        acc_ref[0] = 0.0
        acc_ref[1] = 0.0

    acc_ref[0] += s
    acc_ref[1] += c

    @pl.when(i == n - 1)
    def _fini():
        out_ref[0] = acc_ref[0]
        out_ref[1] = acc_ref[1]


def _sc_body(p_hbm, t_hbm, out_hbm,
             pbuf0, pbuf1, tbuf0, tbuf1, obuf, sp0, sp1, st0, st1):
    cc = lax.axis_index("c")
    ss = lax.axis_index("s")
    wid = ss * _SC_NC + cc
    row0 = _TC_ROWS + wid * _ROWS_W
    pbufs = (pbuf0, pbuf1)
    tbufs = (tbuf0, tbuf1)
    psems = (sp0, sp1)
    tsems = (st0, st1)

    def copies(slot, ci):
        r = row0 + ci * _STRIP
        return (
            pltpu.make_async_copy(p_hbm.at[pl.ds(r, _STRIP)], pbufs[slot],
                                  psems[slot]),
            pltpu.make_async_copy(t_hbm.at[pl.ds(r, _STRIP)], tbufs[slot],
                                  tsems[slot]),
        )

    def start(slot, ci):
        cp, ct = copies(slot, ci)
        cp.start()
        ct.start()

    def wait(slot, ci):
        cp, ct = copies(slot, ci)
        cp.wait()
        ct.wait()

    zf = jnp.zeros((_LANES,), jnp.float32)

    def chunk_sums(slot, s_accs, c_accs):
        pb = pbufs[slot]
        tb = tbufs[slot]

        def body(i, carry):
            s_a, c_a = carry
            col = i * _LANES
            s_l = list(s_a)
            c_l = list(c_a)
            for r in range(_STRIP):
                pv = pb[r, pl.ds(col, _LANES)]
                tv = tb[r, pl.ds(col, _LANES)]
                m = tv != 0.0
                dm = jnp.where(m, pv - tv, 0.0)
                s_l[r] = s_l[r] + dm * dm
                c_l[r] = c_l[r] + jnp.where(m, 1.0, 0.0)
            return tuple(s_l), tuple(c_l)

        return lax.fori_loop(0, _COLS // _LANES, body, (s_accs, c_accs))

    s_accs = (zf,) * _STRIP
    c_accs = (zf,) * _STRIP

    start(0, 0)

    def outer(g, carry):
        s_a, c_a = carry
        for b in range(2):
            ci = g + b
            nxt = ci + 1

            @pl.when(nxt < _NCHUNK)
            def _():
                start((b + 1) % 2, nxt)

            wait(b, ci)
            s_a, c_a = chunk_sums(b, s_a, c_a)
        return s_a, c_a

    s_accs, c_accs = lax.fori_loop(0, _NCHUNK // 2,
                                   lambda g, cr: outer(g * 2, cr),
                                   (s_accs, c_accs))

    s_tot = s_accs[0]
    c_tot = c_accs[0]
    for r in range(1, _STRIP):
        s_tot = s_tot + s_accs[r]
        c_tot = c_tot + c_accs[r]
    obuf[0, pl.ds(0, _LANES)] = s_tot
    obuf[1, pl.ds(0, _LANES)] = c_tot
    pltpu.sync_copy(obuf, out_hbm.at[wid])


_sc_call = functools.partial(
    pl.kernel,
    out_type=jax.ShapeDtypeStruct((_SC_WORKERS, _STRIP, 128), jnp.float32),
    mesh=plsc.VectorSubcoreMesh(core_axis_name="c", subcore_axis_name="s"),
    scratch_types=[
        pltpu.VMEM((_STRIP, _COLS), jnp.float32),
        pltpu.VMEM((_STRIP, _COLS), jnp.float32),
        pltpu.VMEM((_STRIP, _COLS), jnp.float32),
        pltpu.VMEM((_STRIP, _COLS), jnp.float32),
        pltpu.VMEM((_STRIP, 128), jnp.float32),
        pltpu.SemaphoreType.DMA,
        pltpu.SemaphoreType.DMA,
        pltpu.SemaphoreType.DMA,
        pltpu.SemaphoreType.DMA,
    ],
    compiler_params=pltpu.CompilerParams(use_tc_tiling_on_sc=True),
)(_sc_body)


def kernel(prediction, target):
    p2 = prediction.reshape(_ROWS, _COLS)
    t2 = target.reshape(_ROWS, _COLS)

    sc_out = _sc_call(p2, t2)

    tc_out = pl.pallas_call(
        _tc_kernel,
        grid=(_TC_ROWS // _TC_BLOCK_ROWS,),
        in_specs=[
            pl.BlockSpec((_TC_BLOCK_ROWS, _COLS), lambda i: (i, 0)),
            pl.BlockSpec((_TC_BLOCK_ROWS, _COLS), lambda i: (i, 0)),
        ],
        out_specs=pl.BlockSpec(memory_space=pltpu.SMEM),
        out_shape=jax.ShapeDtypeStruct((2,), jnp.float32),
        scratch_shapes=[pltpu.SMEM((2,), jnp.float32)],
    )(p2, t2)

    s = tc_out[0] + jnp.sum(sc_out[:, 0, :_LANES])
    c = tc_out[1] + jnp.sum(sc_out[:, 1, :_LANES])
    return s / c


# speedup vs baseline: 3.4226x; 1.2085x over previous
import jax
import jax.numpy as jnp
from jax.experimental import pallas as pl
from jax.experimental.pallas import tpu as pltpu

_ROWS = 2 * 8192
_COLS = 2048
_BLOCK_ROWS = 1024


def _mse_kernel(pl_ref, pr_ref, tl_ref, tr_ref, out_ref, acc_ref):
    i = pl.program_id(0)
    n = pl.num_programs(0)
    s = 0.0
    c = 0.0
    for p_ref, t_ref in ((pl_ref, tl_ref), (pr_ref, tr_ref)):
        p = p_ref[...]
        t = t_ref[...]
        d = p - t
        mask = t != 0.0
        s += jnp.sum(jnp.where(mask, d * d, 0.0))
        c += jnp.sum(jnp.where(mask, 1.0, 0.0))

    @pl.when(i == 0)
    def _init():
        acc_ref[0] = 0.0
        acc_ref[1] = 0.0

    acc_ref[0] += s
    acc_ref[1] += c

    @pl.when(i == n - 1)
    def _fini():
        out_ref[0] = acc_ref[0] / acc_ref[1]


def kernel(prediction, target):
    p = prediction.reshape(_ROWS, _COLS)
    t = target.reshape(_ROWS, _COLS)
    grid = _ROWS // _BLOCK_ROWS
    out = pl.pallas_call(
        _mse_kernel,
        grid=(grid,),
        in_specs=[
            pl.BlockSpec((_BLOCK_ROWS, _COLS // 2), lambda i: (i, 0)),
            pl.BlockSpec((_BLOCK_ROWS, _COLS // 2), lambda i: (i, 1)),
            pl.BlockSpec((_BLOCK_ROWS, _COLS // 2), lambda i: (i, 0)),
            pl.BlockSpec((_BLOCK_ROWS, _COLS // 2), lambda i: (i, 1)),
        ],
        out_specs=pl.BlockSpec(memory_space=pltpu.SMEM),
        out_shape=jax.ShapeDtypeStruct((1,), jnp.float32),
        scratch_shapes=[pltpu.SMEM((2,), jnp.float32)],
    )(p, p, t, t)
    return out[0]
